# Initial kernel scaffold; baseline (speedup 1.0000x reference)
import jax
import jax.numpy as jnp
from jax.experimental import pallas as pl


def kernel(h, rel_x, r_feat, final_x, batch, mask_ligand, edge_index, xk_W1, xk_b1, xk_g, xk_be, xk_W2, xk_b2, xv_W1, xv_b1, xv_g, xv_be, xv_W2, xv_b2, xq_W1, xq_b1, xq_g, xq_be, xq_W2, xq_b2):
    raise NotImplementedError("write your pallas kernel here")


# speedup vs baseline: 21.4259x; 21.4259x over previous
import functools

import jax
import jax.numpy as jnp
import numpy as np
from jax import lax
from jax.experimental import pallas as pl
from jax.experimental.pallas import tpu as pltpu
from jax.experimental.pallas import tpu_sc as plsc

N = 10000
E = 320000
HID = 128
NH = 16
HD = 8
NC = 2
NS = 16
NW = NC * NS
EPT = E // NW
CHUNK = 80
NCHUNK = EPT // CHUNK
NP = 10240
ROWS_PT = NP // NS
ZCH = 128
BE = 512
BN = 1000

_mesh = plsc.VectorSubcoreMesh(core_axis_name="c", subcore_axis_name="s")


def _ln(x, g, be):
    mu = jnp.mean(x, axis=-1, keepdims=True)
    var = jnp.mean((x - mu) * (x - mu), axis=-1, keepdims=True)
    return (x - mu) / jnp.sqrt(var + 1e-5) * g + be


def _node_body(h_ref, m_ref, ws_ref, wd_ref, wq1_ref, bq1_ref, gq_ref, beq_ref,
               wq2_ref, bq2_ref, s_ref, d_ref):
    hb = h_ref[...]
    s_ref[...] = hb @ ws_ref[...]
    dkv = hb @ wd_ref[...]
    hl = hb * m_ref[...]
    hq = hl @ wq1_ref[...] + bq1_ref[...]
    hq = jax.nn.relu(_ln(hq, gq_ref[...], beq_ref[...]))
    q = hq @ wq2_ref[...] + bq2_ref[...]
    d_ref[...] = jnp.concatenate([dkv, q], axis=-1)


def _node_precompute(h, maskf, w_s, w_d, wq1, bq1, gq, beq, wq2, bq2):
    full = lambda a: pl.BlockSpec(a.shape, lambda i: (0,) * a.ndim)
    return pl.pallas_call(
        _node_body,
        grid=(N // BN,),
        in_specs=[
            pl.BlockSpec((BN, HID), lambda i: (i, 0)),
            pl.BlockSpec((BN, 1), lambda i: (i, 0)),
            full(w_s), full(w_d), full(wq1), full(bq1), full(gq), full(beq),
            full(wq2), full(bq2),
        ],
        out_specs=[
            pl.BlockSpec((BN, 256), lambda i: (i, 0)),
            pl.BlockSpec((BN, 384), lambda i: (i, 0)),
        ],
        out_shape=[
            jax.ShapeDtypeStruct((N, 256), jnp.float32),
            jax.ShapeDtypeStruct((N, 384), jnp.float32),
        ],
    )(h, maskf, w_s, w_d, wq1, bq1, gq, beq, wq2, bq2)


@functools.partial(
    pl.kernel,
    out_type=(
        jax.ShapeDtypeStruct((E, 256), jnp.float32),
        jax.ShapeDtypeStruct((E, 384), jnp.float32),
    ),
    mesh=_mesh,
    scratch_types=[
        pltpu.VMEM((CHUNK,), jnp.int32),
        pltpu.VMEM((CHUNK,), jnp.int32),
        pltpu.VMEM((CHUNK, 256), jnp.float32),
        pltpu.VMEM((CHUNK, 384), jnp.float32),
        pltpu.SemaphoreType.DMA,
        pltpu.SemaphoreType.DMA,
    ],
)
def _sc_gather(s_hbm, d_hbm, src_hbm, dst_hbm, sr_hbm, dr_hbm,
               sidx, didx, sbuf, dbuf, sem1, sem2):
    wid = lax.axis_index("s") * NC + lax.axis_index("c")
    base0 = wid * EPT

    def body(i, carry):
        base = base0 + i * CHUNK
        pltpu.sync_copy(src_hbm.at[pl.ds(base, CHUNK)], sidx)
        pltpu.sync_copy(dst_hbm.at[pl.ds(base, CHUNK)], didx)
        cp1 = pltpu.async_copy(s_hbm.at[sidx], sbuf, sem1)
        cp2 = pltpu.async_copy(d_hbm.at[didx], dbuf, sem2)
        cp1.wait()
        cp2.wait()
        pltpu.sync_copy(sbuf, sr_hbm.at[pl.ds(base, CHUNK)])
        pltpu.sync_copy(dbuf, dr_hbm.at[pl.ds(base, CHUNK)])
        return carry

    lax.fori_loop(0, NCHUNK, body, 0)


def _edge_body(sr_ref, dr_ref, rf_ref, rx_ref, wk1r_ref, wv1r_ref, wk2_ref,
               wv2_ref, bk1_ref, bv1_ref, gk_ref, bek_ref, gv_ref, bev_ref,
               bk2_ref, bv2_ref, out_ref):
    sr = sr_ref[...]
    dr = dr_ref[...]
    rf = rf_ref[...]
    rx = rx_ref[...]
    hdnk = rf @ wk1r_ref[...] + sr[:, 0:128] + dr[:, 0:128] + bk1_ref[...]
    hdnk = jax.nn.relu(_ln(hdnk, gk_ref[...], bek_ref[...]))
    k = hdnk @ wk2_ref[...] + bk2_ref[...]
    hdnv = rf @ wv1r_ref[...] + sr[:, 128:256] + dr[:, 128:256] + bv1_ref[...]
    hdnv = jax.nn.relu(_ln(hdnv, gv_ref[...], bev_ref[...]))
    v16 = hdnv @ wv2_ref[...] + bv2_ref[...]
    qd = dr[:, 256:384]
    lane = lax.broadcasted_iota(jnp.int32, (HID, NH), 0)
    head = lax.broadcasted_iota(jnp.int32, (HID, NH), 1)
    sel = jnp.where(lane // HD == head, 1.0, 0.0).astype(jnp.float32)
    logits = ((qd * k) @ sel) * np.float32(1.0 / np.sqrt(HD))
    ex = jnp.exp(logits)
    w = ex * v16
    z = jnp.zeros_like(qd[:, 0:64])
    out_ref[...] = jnp.concatenate(
        [ex, w * rx[:, 0:1], w * rx[:, 1:2], w * rx[:, 2:3], z], axis=-1)


def _edge_dense(sr, dr, r_feat, relx4, wk1r, wv1r, wk2, wv2, bk1, bv1, gk,
                bek, gv, bev, bk2, bv2):
    full = lambda a: pl.BlockSpec(a.shape, lambda i: (0,) * a.ndim)
    return pl.pallas_call(
        _edge_body,
        grid=(E // BE,),
        in_specs=[
            pl.BlockSpec((BE, 256), lambda i: (i, 0)),
            pl.BlockSpec((BE, 384), lambda i: (i, 0)),
            pl.BlockSpec((BE, 16), lambda i: (i, 0)),
            pl.BlockSpec((BE, 4), lambda i: (i, 0)),
            full(wk1r), full(wv1r), full(wk2), full(wv2), full(bk1), full(bv1),
            full(gk), full(bek), full(gv), full(bev), full(bk2), full(bv2),
        ],
        out_specs=pl.BlockSpec((BE, 128), lambda i: (i, 0)),
        out_shape=jax.ShapeDtypeStruct((E, 128), jnp.float32),
    )(sr, dr, r_feat, relx4, wk1r, wv1r, wk2, wv2, bk1, bv1, gk, bek, gv, bev,
      bk2, bv2)


@functools.partial(
    pl.kernel,
    out_type=jax.ShapeDtypeStruct((NC, NP, 128), jnp.float32),
    mesh=_mesh,
    scratch_types=[
        pltpu.VMEM((CHUNK,), jnp.int32),
        pltpu.VMEM((CHUNK, 128), jnp.float32),
        pltpu.VMEM((ZCH, 128), jnp.float32),
        pltpu.VMEM_SHARED((NP, 128), jnp.float32),
    ],
)
def _sc_scatter(rows_hbm, dst_hbm, out_hbm, didx, rows, zbuf, acc):
    cid = lax.axis_index("c")
    sid = lax.axis_index("s")
    wid = sid * NC + cid
    zero16 = jnp.zeros((16,), jnp.float32)

    def zrow(i, carry):
        for j in range(8):
            zbuf[i, pl.ds(j * 16, 16)] = zero16
        return carry

    lax.fori_loop(0, ZCH, zrow, 0)

    def zcopy(j, carry):
        pltpu.sync_copy(zbuf, acc.at[pl.ds(sid * ROWS_PT + j * ZCH, ZCH)])
        return carry

    lax.fori_loop(0, ROWS_PT // ZCH, zcopy, 0)
    plsc.subcore_barrier()

    def chunk_body(i, carry):
        base = wid * EPT + i * CHUNK
        pltpu.sync_copy(rows_hbm.at[pl.ds(base, CHUNK)], rows)
        pltpu.sync_copy(dst_hbm.at[pl.ds(base, CHUNK)], didx)
        pltpu.sync_copy(rows, acc.at[didx], add=True)
        return carry

    lax.fori_loop(0, NCHUNK, chunk_body, 0)
    plsc.subcore_barrier()
    pltpu.sync_copy(acc.at[pl.ds(sid * ROWS_PT, ROWS_PT)],
                    out_hbm.at[cid, pl.ds(sid * ROWS_PT, ROWS_PT)])


def _final_body(a0_ref, a1_ref, out_ref):
    a = a0_ref[...] + a1_ref[...]
    inv = 1.0 / (a[:, 0:16] + 1e-16)
    o0 = jnp.sum(a[:, 16:32] * inv, axis=-1, keepdims=True)
    o1 = jnp.sum(a[:, 32:48] * inv, axis=-1, keepdims=True)
    o2 = jnp.sum(a[:, 48:64] * inv, axis=-1, keepdims=True)
    out_ref[...] = jnp.concatenate([o0, o1, o2, o0 * 0.0],
                                   axis=-1) * np.float32(1.0 / NH)


def _final(a0, a1):
    spec = pl.BlockSpec((BN, 128), lambda i: (i, 0))
    return pl.pallas_call(
        _final_body,
        grid=(N // BN,),
        in_specs=[spec, spec],
        out_specs=pl.BlockSpec((BN, 4), lambda i: (i, 0)),
        out_shape=jax.ShapeDtypeStruct((N, 4), jnp.float32),
    )(a0, a1)


def kernel(h, rel_x, r_feat, final_x, batch, mask_ligand, edge_index,
           xk_W1, xk_b1, xk_g, xk_be, xk_W2, xk_b2,
           xv_W1, xv_b1, xv_g, xv_be, xv_W2, xv_b2,
           xq_W1, xq_b1, xq_g, xq_be, xq_W2, xq_b2):
    src = edge_index[0]
    dst = edge_index[1]
    maskf = mask_ligand.astype(jnp.float32).reshape(N, 1)
    relx4 = jnp.concatenate(
        [rel_x, jnp.zeros((E, 1), jnp.float32)], axis=-1)
    w_s = jnp.concatenate([xk_W1[16:144], xv_W1[16:144]], axis=1)
    w_d = jnp.concatenate([xk_W1[144:272], xv_W1[144:272]], axis=1)
    row = lambda b: b.reshape(1, -1)

    s_tab, d_tab = _node_precompute(
        h, maskf, w_s, w_d, xq_W1, row(xq_b1), row(xq_g), row(xq_be), xq_W2,
        row(xq_b2))
    sr, dr = _sc_gather(s_tab, d_tab, src, dst)
    rows = _edge_dense(sr, dr, r_feat, relx4, xk_W1[0:16], xv_W1[0:16],
                       xk_W2, xv_W2, row(xk_b1), row(xv_b1), row(xk_g),
                       row(xk_be), row(xv_g), row(xv_be), row(xk_b2),
                       row(xv_b2))
    parts = _sc_scatter(rows, dst)
    out4 = _final(parts[0], parts[1])
    return out4[:, 0:3]
